# Initial kernel scaffold; baseline (speedup 1.0000x reference)
import jax
import jax.numpy as jnp
from jax.experimental import pallas as pl


def kernel(f_G, u_G, v_G, id_T, u_T, v_T, embeddings, W1_G, W2_G, W3_G, b1_G, U1_G, U2_G, b2_G, W1_T, W2_T, W3_T, b1_T, U1_T, U2_T, b2_T):
    raise NotImplementedError("write your pallas kernel here")


# speedup vs baseline: 1.6349x; 1.6349x over previous
import functools

import jax
import jax.numpy as jnp
from jax import lax
from jax.experimental import pallas as pl
from jax.experimental.pallas import tpu as pltpu
from jax.experimental.pallas import tpu_sc as plsc

D = 128
NC = 2
NS = 16
NW = NC * NS
CH = 128

N_G, EH_G, N_T, EH_T, VOCAB = 10000, 160000, 5000, 5000, 800
N_ITERS = 4


def _pad_to(x, n, val=0):
    pad = [(0, n - x.shape[0])] + [(0, 0)] * (x.ndim - 1)
    return jnp.pad(x, pad, constant_values=val)


def _round_up(n, m):
    return (n + m - 1) // m * m


def _sc_gather(table, idx2d, out_rows):
    v_tab, _ = table.shape
    n_chunks_tot = idx2d.shape[0]
    n_chunks = n_chunks_tot // NW
    mesh = plsc.VectorSubcoreMesh(core_axis_name="c", subcore_axis_name="s")

    @functools.partial(
        pl.kernel,
        out_type=jax.ShapeDtypeStruct((out_rows, D), jnp.float32),
        mesh=mesh,
        scratch_types=[
            pltpu.VMEM((n_chunks, CH), jnp.int32),
            pltpu.VMEM((CH, D), jnp.float32),
        ],
    )
    def k(table_hbm, idx_hbm, out_hbm, idx_v, rows_v):
        wid = lax.axis_index("s") * NC + lax.axis_index("c")
        base_c = wid * n_chunks
        pltpu.sync_copy(idx_hbm.at[pl.ds(base_c, n_chunks)], idx_v)

        @pl.loop(0, n_chunks)
        def _(j):
            pltpu.sync_copy(table_hbm.at[idx_v.at[j]], rows_v)
            pltpu.sync_copy(rows_v, out_hbm.at[pl.ds((base_c + j) * CH, CH)])

    return k(table, idx2d)


def _sc_pre_gather(a_tab, b_tab, u2d, v2d, ehp):
    n_chunks_tot = u2d.shape[0]
    n_chunks = n_chunks_tot // NW
    mesh = plsc.VectorSubcoreMesh(core_axis_name="c", subcore_axis_name="s")

    @functools.partial(
        pl.kernel,
        out_type=jax.ShapeDtypeStruct((2 * ehp, D), jnp.float32),
        mesh=mesh,
        scratch_types=[
            pltpu.VMEM((n_chunks, CH), jnp.int32),
            pltpu.VMEM((n_chunks, CH), jnp.int32),
            pltpu.VMEM((CH, D), jnp.float32),
            pltpu.VMEM((CH, D), jnp.float32),
            pltpu.VMEM((CH, D), jnp.float32),
            pltpu.VMEM((CH, D), jnp.float32),
        ],
    )
    def k(a_hbm, b_hbm, u_hbm, v_hbm, out_hbm, u_v, v_v, au, bv, av, bu):
        wid = lax.axis_index("s") * NC + lax.axis_index("c")
        base_c = wid * n_chunks
        pltpu.sync_copy(u_hbm.at[pl.ds(base_c, n_chunks)], u_v)
        pltpu.sync_copy(v_hbm.at[pl.ds(base_c, n_chunks)], v_v)

        @pl.loop(0, n_chunks)
        def _(j):
            pltpu.sync_copy(a_hbm.at[u_v.at[j]], au)
            pltpu.sync_copy(b_hbm.at[v_v.at[j]], bv)
            pltpu.sync_copy(a_hbm.at[v_v.at[j]], av)
            pltpu.sync_copy(b_hbm.at[u_v.at[j]], bu)

            @pl.loop(0, CH)
            def _(r):
                for s in range(D // 16):
                    sl = pl.ds(s * 16, 16)
                    x = au[r, sl] + bv[r, sl]
                    y = av[r, sl] + bu[r, sl]
                    au[r, sl] = x
                    av[r, sl] = y

            pltpu.sync_copy(au, out_hbm.at[pl.ds((base_c + j) * CH, CH)])
            pltpu.sync_copy(av, out_hbm.at[pl.ds(ehp + (base_c + j) * CH, CH)])

    return k(a_tab, b_tab, u2d, v2d)


def _sc_scatter_add(data, idx2d, n_pad):
    e_pad = data.shape[0]
    n_chunks = e_pad // CH // NW
    rows_per_tile = n_pad // NS
    mesh = plsc.VectorSubcoreMesh(core_axis_name="c", subcore_axis_name="s")

    @functools.partial(
        pl.kernel,
        out_type=jax.ShapeDtypeStruct((NC, n_pad, D), jnp.float32),
        mesh=mesh,
        scratch_types=[
            pltpu.VMEM((n_chunks, CH), jnp.int32),
            pltpu.VMEM((CH, D), jnp.float32),
            pltpu.VMEM((8, D), jnp.float32),
            pltpu.VMEM_SHARED((n_pad, D), jnp.float32),
        ],
    )
    def k(data_hbm, idx_hbm, out_hbm, idx_v, data_v, zero_v, agg_s):
        cid = lax.axis_index("c")
        sid = lax.axis_index("s")
        wid = sid * NC + cid
        base_c = wid * n_chunks

        @pl.loop(0, 8)
        def _(r):
            for s in range(D // 16):
                zero_v[r, pl.ds(s * 16, 16)] = jnp.zeros((16,), jnp.float32)

        @pl.loop(0, rows_per_tile // 8)
        def _(r):
            pltpu.sync_copy(
                zero_v, agg_s.at[pl.ds(sid * rows_per_tile + r * 8, 8)])

        plsc.subcore_barrier()

        pltpu.sync_copy(idx_hbm.at[pl.ds(base_c, n_chunks)], idx_v)

        @pl.loop(0, n_chunks)
        def _(j):
            pltpu.sync_copy(data_hbm.at[pl.ds((base_c + j) * CH, CH)], data_v)
            pltpu.sync_copy(data_v, agg_s.at[idx_v.at[j]], add=True)

        plsc.subcore_barrier()
        sl = pl.ds(sid * rows_per_tile, rows_per_tile)
        pltpu.sync_copy(agg_s.at[sl], out_hbm.at[cid, sl])

    return k(data, idx2d)


def _tc_node_ab(f_pad, w1, w2, b1):
    np_, _ = f_pad.shape
    blk = np_ // 16
    grid = 16

    def body(f_ref, w1_ref, w2_ref, b1_ref, a_ref, b_ref):
        f = f_ref[...]
        a_ref[...] = f @ w1_ref[...] + b1_ref[...]
        b_ref[...] = f @ w2_ref[...]

    return pl.pallas_call(
        body,
        grid=(grid,),
        in_specs=[
            pl.BlockSpec((blk, D), lambda i: (i, 0)),
            pl.BlockSpec((D, D), lambda i: (0, 0)),
            pl.BlockSpec((D, D), lambda i: (0, 0)),
            pl.BlockSpec((1, D), lambda i: (0, 0)),
        ],
        out_specs=[
            pl.BlockSpec((blk, D), lambda i: (i, 0)),
            pl.BlockSpec((blk, D), lambda i: (i, 0)),
        ],
        out_shape=[
            jax.ShapeDtypeStruct((np_, D), jnp.float32),
            jax.ShapeDtypeStruct((np_, D), jnp.float32),
        ],
    )(f_pad, w1, w2, b1.reshape(1, D))


def _tc_relu(x):
    n = x.shape[0]
    blk = 4096
    grid = n // blk

    def body(x_ref, o_ref):
        o_ref[...] = jnp.maximum(x_ref[...], 0.0)

    return pl.pallas_call(
        body,
        grid=(grid,),
        in_specs=[pl.BlockSpec((blk, D), lambda i: (i, 0))],
        out_specs=pl.BlockSpec((blk, D), lambda i: (i, 0)),
        out_shape=jax.ShapeDtypeStruct((n, D), jnp.float32),
    )(x)


def _tc_combine(partials):
    np_ = partials.shape[1]
    blk = np_ // 16

    def body(p_ref, o_ref):
        o_ref[...] = p_ref[0] + p_ref[1]

    return pl.pallas_call(
        body,
        grid=(16,),
        in_specs=[pl.BlockSpec((2, blk, D), lambda i: (0, i, 0))],
        out_specs=pl.BlockSpec((blk, D), lambda i: (i, 0)),
        out_shape=jax.ShapeDtypeStruct((np_, D), jnp.float32),
    )(partials)


def _tc_msg_update(gathered, msg_old, pre, w3):
    ep = gathered.shape[0]
    blk = 1024
    nb = ep // blk
    nbh = nb // 2

    def body(g_ref, mr_ref, p_ref, w3_ref, o_ref):
        s = g_ref[...] - mr_ref[...]
        o_ref[...] = jnp.maximum(p_ref[...] + s @ w3_ref[...], 0.0)

    return pl.pallas_call(
        body,
        grid=(nb,),
        in_specs=[
            pl.BlockSpec((blk, D), lambda i: (i, 0)),
            pl.BlockSpec((blk, D), lambda i: ((i + nbh) % nb, 0)),
            pl.BlockSpec((blk, D), lambda i: (i, 0)),
            pl.BlockSpec((D, D), lambda i: (0, 0)),
        ],
        out_specs=pl.BlockSpec((blk, D), lambda i: (i, 0)),
        out_shape=jax.ShapeDtypeStruct((ep, D), jnp.float32),
    )(gathered, msg_old, pre, w3)


def _tc_readout(f, ns, u1, u2, b2):
    n = f.shape[0]
    blk = 1000
    grid = n // blk

    def body(f_ref, ns_ref, u1_ref, u2_ref, b2_ref, o_ref):
        o_ref[...] = jnp.maximum(
            f_ref[...] @ u1_ref[...] + ns_ref[...] @ u2_ref[...] + b2_ref[...], 0.0
        )

    return pl.pallas_call(
        body,
        grid=(grid,),
        in_specs=[
            pl.BlockSpec((blk, D), lambda i: (i, 0)),
            pl.BlockSpec((blk, D), lambda i: (i, 0)),
            pl.BlockSpec((D, D), lambda i: (0, 0)),
            pl.BlockSpec((D, D), lambda i: (0, 0)),
            pl.BlockSpec((1, D), lambda i: (0, 0)),
        ],
        out_specs=pl.BlockSpec((blk, D), lambda i: (i, 0)),
        out_shape=jax.ShapeDtypeStruct((n, D), jnp.float32),
    )(f, ns, u1, u2, b2.reshape(1, D))


def _encode(f, u, v, w1, w2, w3, b1, u1, u2, b2, n, n_iters):
    eh = u.shape[0]
    ehp = _round_up(eh, NW * CH)
    ep = 2 * ehp
    n_pad = _round_up(n + 1, NS * 8)

    u_p = _pad_to(u, ehp, n)
    v_p = _pad_to(v, ehp, n)
    src = jnp.concatenate([u_p, v_p]).reshape(ep // CH, CH)
    dst = jnp.concatenate([v_p, u_p]).reshape(ep // CH, CH)
    u2d = u_p.reshape(ehp // CH, CH)
    v2d = v_p.reshape(ehp // CH, CH)
    f_pad = _pad_to(f, n_pad)

    a_tab, b_tab = _tc_node_ab(f_pad, w1, w2, b1)
    pre = _sc_pre_gather(a_tab, b_tab, u2d, v2d, ehp)
    msg = _tc_relu(pre)
    for _ in range(n_iters - 1):
        partials = _sc_scatter_add(msg, dst, n_pad)
        agg = _tc_combine(partials)
        gathered = _sc_gather(agg, src, ep)
        msg = _tc_msg_update(gathered, msg, pre, w3)
    partials = _sc_scatter_add(msg, dst, n_pad)
    ns = _tc_combine(partials)[:n]
    return _tc_readout(f, ns, u1, u2, b2)


def kernel(f_G, u_G, v_G, id_T, u_T, v_T, embeddings, W1_G, W2_G, W3_G, b1_G,
           U1_G, U2_G, b2_G, W1_T, W2_T, W3_T, b1_T, U1_T, U2_T, b2_T):
    x_G = _encode(f_G, u_G, v_G, W1_G, W2_G, W3_G, b1_G, U1_G, U2_G, b2_G,
                  N_G, N_ITERS)
    idp = _round_up(N_T, NW * CH)
    id2d = _pad_to(id_T, idp).reshape(idp // CH, CH)
    f_T = _sc_gather(embeddings, id2d, idp)[:N_T]
    x_T = _encode(f_T, u_T, v_T, W1_T, W2_T, W3_T, b1_T, U1_T, U2_T, b2_T,
                  N_T, N_ITERS)
    return x_G, x_T


# speedup vs baseline: 1.6589x; 1.0147x over previous
import functools

import jax
import jax.numpy as jnp
from jax import lax
from jax.experimental import pallas as pl
from jax.experimental.pallas import tpu as pltpu
from jax.experimental.pallas import tpu_sc as plsc

D = 128
NC = 2
NS = 16
NW = NC * NS

N_G, EH_G, N_T, EH_T, VOCAB = 10000, 160000, 5000, 5000, 800
N_ITERS = 4


def _pad_to(x, n, val=0):
    pad = [(0, n - x.shape[0])] + [(0, 0)] * (x.ndim - 1)
    return jnp.pad(x, pad, constant_values=val)


def _round_up(n, m):
    return (n + m - 1) // m * m


def _sc_gather(table, idx2d, out_rows):
    ch = idx2d.shape[1]
    n_chunks = idx2d.shape[0] // NW
    nb = min(4, n_chunks)
    mesh = plsc.VectorSubcoreMesh(core_axis_name="c", subcore_axis_name="s")

    @functools.partial(
        pl.kernel,
        out_type=jax.ShapeDtypeStruct((out_rows, D), jnp.float32),
        mesh=mesh,
        scratch_types=(
            [pltpu.VMEM((n_chunks, ch), jnp.int32)]
            + [pltpu.VMEM((ch, D), jnp.float32) for _ in range(nb)]
            + [pltpu.SemaphoreType.DMA for _ in range(2 * nb)]
        ),
    )
    def k(table_hbm, idx_hbm, out_hbm, idx_v, *rest):
        bufs, semi, semo = rest[:nb], rest[nb:2 * nb], rest[2 * nb:3 * nb]
        wid = lax.axis_index("s") * NC + lax.axis_index("c")
        base_c = wid * n_chunks
        pltpu.sync_copy(idx_hbm.at[pl.ds(base_c, n_chunks)], idx_v)
        for b in range(nb):
            pltpu.async_copy(table_hbm.at[idx_v.at[b]], bufs[b], semi[b])

        @pl.loop(0, n_chunks, step=nb)
        def _(g):
            for b in range(nb):
                j = g + b
                pltpu.make_async_copy(
                    table_hbm.at[idx_v.at[j]], bufs[b], semi[b]).wait()
                pltpu.async_copy(
                    bufs[b], out_hbm.at[pl.ds((base_c + j) * ch, ch)], semo[b])
            for b in range(nb):
                j = g + b
                pltpu.make_async_copy(
                    bufs[b], out_hbm.at[pl.ds((base_c + j) * ch, ch)],
                    semo[b]).wait()

                @pl.when(j + nb < n_chunks)
                def _():
                    pltpu.async_copy(
                        table_hbm.at[idx_v.at[j + nb]], bufs[b], semi[b])

    return k(table, idx2d)


def _sc_pre_gather(ab_tab, u2d, v2d, ehp):
    ch = u2d.shape[1]
    n_chunks = u2d.shape[0] // NW
    nb = min(2, n_chunks)
    mesh = plsc.VectorSubcoreMesh(core_axis_name="c", subcore_axis_name="s")

    @functools.partial(
        pl.kernel,
        out_type=jax.ShapeDtypeStruct((2 * ehp, D), jnp.float32),
        mesh=mesh,
        scratch_types=(
            [pltpu.VMEM((n_chunks, ch), jnp.int32) for _ in range(2)]
            + [pltpu.VMEM((ch, 2 * D), jnp.float32) for _ in range(2 * nb)]
            + [pltpu.VMEM((ch, D), jnp.float32) for _ in range(2 * nb)]
            + [pltpu.SemaphoreType.DMA for _ in range(3 * nb)]
        ),
    )
    def k(ab_hbm, u_hbm, v_hbm, out_hbm, u_v, v_v, *rest):
        abu = rest[:nb]
        abv = rest[nb:2 * nb]
        pf = rest[2 * nb:3 * nb]
        pb = rest[3 * nb:4 * nb]
        semu = rest[4 * nb:5 * nb]
        semv = rest[5 * nb:6 * nb]
        semo = rest[6 * nb:7 * nb]
        wid = lax.axis_index("s") * NC + lax.axis_index("c")
        base_c = wid * n_chunks
        pltpu.sync_copy(u_hbm.at[pl.ds(base_c, n_chunks)], u_v)
        pltpu.sync_copy(v_hbm.at[pl.ds(base_c, n_chunks)], v_v)
        for b in range(nb):
            pltpu.async_copy(ab_hbm.at[u_v.at[b]], abu[b], semu[b])
            pltpu.async_copy(ab_hbm.at[v_v.at[b]], abv[b], semv[b])

        @pl.loop(0, n_chunks, step=nb)
        def _(g):
            for b in range(nb):
                j = g + b
                pltpu.make_async_copy(
                    ab_hbm.at[u_v.at[j]], abu[b], semu[b]).wait()
                pltpu.make_async_copy(
                    ab_hbm.at[v_v.at[j]], abv[b], semv[b]).wait()

                @pl.loop(0, ch)
                def _(r):
                    for s in range(D // 16):
                        sa = pl.ds(s * 16, 16)
                        sb = pl.ds(D + s * 16, 16)
                        pf[b][r, sa] = abu[b][r, sa] + abv[b][r, sb]
                        pb[b][r, sa] = abv[b][r, sa] + abu[b][r, sb]

                pltpu.async_copy(
                    pf[b], out_hbm.at[pl.ds((base_c + j) * ch, ch)], semo[b])
                pltpu.async_copy(
                    pb[b], out_hbm.at[pl.ds(ehp + (base_c + j) * ch, ch)],
                    semo[b])
            for b in range(nb):
                j = g + b
                pltpu.make_async_copy(
                    pf[b], out_hbm.at[pl.ds((base_c + j) * ch, ch)],
                    semo[b]).wait()
                pltpu.make_async_copy(
                    pb[b], out_hbm.at[pl.ds(ehp + (base_c + j) * ch, ch)],
                    semo[b]).wait()

                @pl.when(j + nb < n_chunks)
                def _():
                    pltpu.async_copy(ab_hbm.at[u_v.at[j + nb]], abu[b], semu[b])
                    pltpu.async_copy(ab_hbm.at[v_v.at[j + nb]], abv[b], semv[b])

    return k(ab_tab, u2d, v2d)


def _sc_scatter_add(data, idx2d, zeros, n_pad, row_off=0):
    ch = idx2d.shape[1]
    n_chunks = idx2d.shape[0] // NW
    nb = min(2, n_chunks)
    rpt = n_pad // NS
    mesh = plsc.VectorSubcoreMesh(core_axis_name="c", subcore_axis_name="s")

    @functools.partial(
        pl.kernel,
        out_type=jax.ShapeDtypeStruct((NC, n_pad, D), jnp.float32),
        mesh=mesh,
        scratch_types=(
            [pltpu.VMEM((n_chunks, ch), jnp.int32)]
            + [pltpu.VMEM((ch, D), jnp.float32) for _ in range(nb)]
            + [pltpu.VMEM_SHARED((n_pad, D), jnp.float32)]
            + [pltpu.SemaphoreType.DMA for _ in range(2 * nb)]
        ),
    )
    def k(data_hbm, idx_hbm, zeros_hbm, out_hbm, idx_v, *rest):
        bufs = rest[:nb]
        agg_s = rest[nb]
        semi, semo = rest[nb + 1:2 * nb + 1], rest[2 * nb + 1:3 * nb + 1]
        cid = lax.axis_index("c")
        sid = lax.axis_index("s")
        wid = sid * NC + cid
        base_c = wid * n_chunks
        my = pl.ds(sid * rpt, rpt)
        pltpu.sync_copy(zeros_hbm, agg_s.at[my])
        pltpu.sync_copy(idx_hbm.at[pl.ds(base_c, n_chunks)], idx_v)
        plsc.subcore_barrier()

        def in_cp(j, b, sem):
            return pltpu.make_async_copy(
                data_hbm.at[pl.ds(row_off + (base_c + j) * ch, ch)],
                bufs[b], sem)

        for b in range(nb):
            in_cp(b, b, semi[b]).start()

        @pl.loop(0, n_chunks, step=nb)
        def _(g):
            for b in range(nb):
                j = g + b
                in_cp(j, b, semi[b]).wait()
                pltpu.async_copy(bufs[b], agg_s.at[idx_v.at[j]], semo[b],
                                 add=True)
            for b in range(nb):
                j = g + b
                pltpu.make_async_copy(
                    bufs[b], agg_s.at[idx_v.at[j]], semo[b]).wait()

                @pl.when(j + nb < n_chunks)
                def _():
                    in_cp(j + nb, b, semi[b]).start()

        plsc.subcore_barrier()
        pltpu.sync_copy(agg_s.at[my], out_hbm.at[cid, my])

    return k(data, idx2d, zeros)


def _tc_node_ab(f_pad, w1, w2, b1):
    np_, _ = f_pad.shape
    blk = np_ // 16

    def body(f_ref, w1_ref, w2_ref, b1_ref, ab_ref):
        f = f_ref[...]
        ab_ref[:, :D] = f @ w1_ref[...] + b1_ref[...]
        ab_ref[:, D:] = f @ w2_ref[...]

    return pl.pallas_call(
        body,
        grid=(16,),
        in_specs=[
            pl.BlockSpec((blk, D), lambda i: (i, 0)),
            pl.BlockSpec((D, D), lambda i: (0, 0)),
            pl.BlockSpec((D, D), lambda i: (0, 0)),
            pl.BlockSpec((1, D), lambda i: (0, 0)),
        ],
        out_specs=pl.BlockSpec((blk, 2 * D), lambda i: (i, 0)),
        out_shape=jax.ShapeDtypeStruct((np_, 2 * D), jnp.float32),
    )(f_pad, w1, w2, b1.reshape(1, D))


def _tc_relu(x):
    n = x.shape[0]
    blk = 4096
    grid = n // blk

    def body(x_ref, o_ref):
        o_ref[...] = jnp.maximum(x_ref[...], 0.0)

    return pl.pallas_call(
        body,
        grid=(grid,),
        in_specs=[pl.BlockSpec((blk, D), lambda i: (i, 0))],
        out_specs=pl.BlockSpec((blk, D), lambda i: (i, 0)),
        out_shape=jax.ShapeDtypeStruct((n, D), jnp.float32),
    )(x)


def _tc_combine(partials):
    np_ = partials.shape[1]
    blk = np_ // 16

    def body(p_ref, o_ref):
        o_ref[...] = p_ref[0] + p_ref[1]

    return pl.pallas_call(
        body,
        grid=(16,),
        in_specs=[pl.BlockSpec((2, blk, D), lambda i: (0, i, 0))],
        out_specs=pl.BlockSpec((blk, D), lambda i: (i, 0)),
        out_shape=jax.ShapeDtypeStruct((np_, D), jnp.float32),
    )(partials)


def _tc_msg_update(gathered, msg_old, pre, w3):
    ep = gathered.shape[0]
    blk = 1024
    nb = ep // blk
    nbh = nb // 2

    def body(g_ref, mr_ref, p_ref, w3_ref, o_ref):
        s = g_ref[...] - mr_ref[...]
        o_ref[...] = jnp.maximum(p_ref[...] + s @ w3_ref[...], 0.0)

    return pl.pallas_call(
        body,
        grid=(nb,),
        in_specs=[
            pl.BlockSpec((blk, D), lambda i: (i, 0)),
            pl.BlockSpec((blk, D), lambda i: ((i + nbh) % nb, 0)),
            pl.BlockSpec((blk, D), lambda i: (i, 0)),
            pl.BlockSpec((D, D), lambda i: (0, 0)),
        ],
        out_specs=pl.BlockSpec((blk, D), lambda i: (i, 0)),
        out_shape=jax.ShapeDtypeStruct((ep, D), jnp.float32),
    )(gathered, msg_old, pre, w3)


def _tc_readout(f, ns, u1, u2, b2):
    n = f.shape[0]
    blk = 1000
    grid = n // blk

    def body(f_ref, ns_ref, u1_ref, u2_ref, b2_ref, o_ref):
        o_ref[...] = jnp.maximum(
            f_ref[...] @ u1_ref[...] + ns_ref[...] @ u2_ref[...] + b2_ref[...],
            0.0)

    return pl.pallas_call(
        body,
        grid=(grid,),
        in_specs=[
            pl.BlockSpec((blk, D), lambda i: (i, 0)),
            pl.BlockSpec((blk, D), lambda i: (i, 0)),
            pl.BlockSpec((D, D), lambda i: (0, 0)),
            pl.BlockSpec((D, D), lambda i: (0, 0)),
            pl.BlockSpec((1, D), lambda i: (0, 0)),
        ],
        out_specs=pl.BlockSpec((blk, D), lambda i: (i, 0)),
        out_shape=jax.ShapeDtypeStruct((n, D), jnp.float32),
    )(f, ns, u1, u2, b2.reshape(1, D))


def _encode(f, u, v, w1, w2, w3, b1, u1, u2, b2, n, n_iters):
    eh = u.shape[0]
    ehp = _round_up(eh, NW * 128)
    ep = 2 * ehp
    n_pad = _round_up(n + 1, NS * 8)
    rpt = n_pad // NS

    u_p = _pad_to(u, ehp, n)
    v_p = _pad_to(v, ehp, n)
    src = jnp.concatenate([u_p, v_p]).reshape(ep // 128, 128)
    dst = jnp.concatenate([v_p, u_p]).reshape(ep // 64, 64)
    u2d = u_p.reshape(ehp // 32, 32)
    v2d = v_p.reshape(ehp // 32, 32)
    f_pad = _pad_to(f, n_pad)
    zeros = jnp.zeros((rpt, D), jnp.float32)

    ab_tab = _tc_node_ab(f_pad, w1, w2, b1)
    pre = _sc_pre_gather(ab_tab, u2d, v2d, ehp)
    msg = _tc_relu(pre)
    for _ in range(n_iters - 1):
        partials = _sc_scatter_add(msg, dst, zeros, n_pad)
        agg = _tc_combine(partials)
        gathered = _sc_gather(agg, src, ep)
        msg = _tc_msg_update(gathered, msg, pre, w3)
    partials = _sc_scatter_add(msg, dst, zeros, n_pad)
    ns = _tc_combine(partials)[:n]
    return _tc_readout(f, ns, u1, u2, b2)


def kernel(f_G, u_G, v_G, id_T, u_T, v_T, embeddings, W1_G, W2_G, W3_G, b1_G,
           U1_G, U2_G, b2_G, W1_T, W2_T, W3_T, b1_T, U1_T, U2_T, b2_T):
    x_G = _encode(f_G, u_G, v_G, W1_G, W2_G, W3_G, b1_G, U1_G, U2_G, b2_G,
                  N_G, N_ITERS)
    idp = _round_up(N_T, NW * 128)
    id2d = _pad_to(id_T, idp).reshape(idp // 128, 128)
    f_T = _sc_gather(embeddings, id2d, idp)[:N_T]
    x_T = _encode(f_T, u_T, v_T, W1_T, W2_T, W3_T, b1_T, U1_T, U2_T, b2_T,
                  N_T, N_ITERS)
    return x_G, x_T
